# baseline (speedup 1.0000x reference)
import functools

import jax
import jax.numpy as jnp
from jax import lax
from jax.experimental import pallas as pl
from jax.experimental.pallas import tpu as pltpu
from jax.experimental.pallas import tpu_sc as plsc

N = 10000
NP = 10240
E = 320000
H = 128
T = 3
L = 3

NUM_CORES = 2
NUM_SUBCORES = 16
NUM_WORKERS = NUM_CORES * NUM_SUBCORES
EPW = E // NUM_WORKERS
CHUNK = 80
CHUNKS = EPW // CHUNK
ROWS_PER_TILE = NP // NUM_SUBCORES
FLUSH_CHUNK = CHUNK


def _make_sc_agg(num_towers: int):
    mesh = plsc.VectorSubcoreMesh(core_axis_name="c", subcore_axis_name="s")

    def body(h_hbm, src_hbm, dst_hbm, zeros_hbm, out_hbm,
             src_blk, dst_blk, rows0, rows1, acc,
             gsem0, gsem1, ssem0, ssem1):
        c = lax.axis_index("c")
        s = lax.axis_index("s")
        wid = c * NUM_SUBCORES + s

        pltpu.sync_copy(src_hbm.at[wid], src_blk)
        pltpu.sync_copy(dst_hbm.at[wid], dst_blk)

        for t in range(num_towers):
            pltpu.sync_copy(zeros_hbm, rows1)
            r0 = s * ROWS_PER_TILE
            for k in range(ROWS_PER_TILE // FLUSH_CHUNK):
                pltpu.sync_copy(
                    rows1,
                    acc.at[pl.ds(r0 + k * FLUSH_CHUNK, FLUSH_CHUNK)])
            plsc.subcore_barrier()

            table = h_hbm.at[t]

            def fire_g(j, rowsbuf, gsem):
                off = pl.multiple_of(j * CHUNK, CHUNK)
                pltpu.async_copy(table.at[src_blk.at[pl.ds(off, CHUNK)]],
                                 rowsbuf, gsem)

            def wait_g(j, rowsbuf, gsem):
                off = pl.multiple_of(j * CHUNK, CHUNK)
                pltpu.make_async_copy(table.at[src_blk.at[pl.ds(off, CHUNK)]],
                                      rowsbuf, gsem).wait()

            def fire_s(j, rowsbuf, ssem):
                pltpu.async_copy(rowsbuf, acc.at[dst_blk.at[j]], ssem,
                                 add=True)

            def wait_s(j, rowsbuf, ssem):
                pltpu.make_async_copy(rowsbuf, acc.at[dst_blk.at[j]],
                                      ssem).wait()

            fire_g(0, rows0, gsem0)
            wait_g(0, rows0, gsem0)
            fire_s(0, rows0, ssem0)
            fire_g(1, rows1, gsem1)

            def pair_body(p, carry):
                k1 = 2 * p + 1
                wait_g(k1, rows1, gsem1)
                fire_s(k1, rows1, ssem1)
                wait_s(k1 - 1, rows0, ssem0)
                fire_g(k1 + 1, rows0, gsem0)
                wait_g(k1 + 1, rows0, gsem0)
                fire_s(k1 + 1, rows0, ssem0)
                wait_s(k1, rows1, ssem1)
                fire_g(k1 + 2, rows1, gsem1)
                return carry

            lax.fori_loop(0, (CHUNKS - 3) // 2, pair_body, 0)

            k = CHUNKS - 2
            wait_g(k, rows1, gsem1)
            fire_s(k, rows1, ssem1)
            wait_s(k - 1, rows0, ssem0)
            fire_g(k + 1, rows0, gsem0)
            wait_g(k + 1, rows0, gsem0)
            fire_s(k + 1, rows0, ssem0)
            wait_s(k, rows1, ssem1)
            wait_s(k + 1, rows0, ssem0)

            plsc.subcore_barrier()

            for k in range(ROWS_PER_TILE // FLUSH_CHUNK):
                off = r0 + k * FLUSH_CHUNK
                pltpu.sync_copy(acc.at[pl.ds(off, FLUSH_CHUNK)], rows0)
                pltpu.sync_copy(rows0,
                                out_hbm.at[c, t, pl.ds(off, FLUSH_CHUNK)])
            plsc.subcore_barrier()

    return pl.kernel(
        body,
        out_type=jax.ShapeDtypeStruct((NUM_CORES, num_towers, NP, H),
                                      jnp.float32),
        mesh=mesh,
        scratch_types=[
            pltpu.VMEM((EPW,), jnp.int32),
            pltpu.VMEM((CHUNKS, CHUNK), jnp.int32),
            pltpu.VMEM((CHUNK, H), jnp.float32),
            pltpu.VMEM((CHUNK, H), jnp.float32),
            pltpu.VMEM_SHARED((NP, H), jnp.float32),
            pltpu.SemaphoreType.DMA,
            pltpu.SemaphoreType.DMA,
            pltpu.SemaphoreType.DMA,
            pltpu.SemaphoreType.DMA,
        ],
    )


_sc_agg_1 = _make_sc_agg(1)
_sc_agg_3 = _make_sc_agg(T)

BN = 1024
GRID = NP // BN


def _mm(a, w):
    return lax.dot_general(a, w, (((1,), (0,)), ((), ())),
                           preferred_element_type=jnp.float32,
                           precision=lax.Precision.HIGHEST)


def _mlp_first_body(scale_ref, x_ref, aggp_ref, w1_ref, b1_ref, w2_ref,
                    b2_ref, out_ref):
    agg = aggp_ref[0] + aggp_ref[1]
    x = x_ref[...]
    for t in range(T):
        u = scale_ref[t] * x + agg
        v = jnp.maximum(_mm(u, w1_ref[t]) + b1_ref[t], 0.0)
        w = jnp.maximum(_mm(v, w2_ref[t]) + b2_ref[t], 0.0)
        out_ref[t] = w


def _mlp_mid_body(scale_ref, h_ref, aggp_ref, w1_ref, b1_ref, w2_ref,
                  b2_ref, out_ref):
    for t in range(T):
        u = scale_ref[t] * h_ref[t] + (aggp_ref[0, t] + aggp_ref[1, t])
        v = jnp.maximum(_mm(u, w1_ref[t]) + b1_ref[t], 0.0)
        w = jnp.maximum(_mm(v, w2_ref[t]) + b2_ref[t], 0.0)
        out_ref[t] = w


_W_SPEC = pl.BlockSpec((T, H, H), lambda i: (0, 0, 0))
_B_SPEC = pl.BlockSpec((T, H), lambda i: (0, 0))
_H3_SPEC = pl.BlockSpec((T, BN, H), lambda i: (0, i, 0))

_mlp_first = pl.pallas_call(
    _mlp_first_body,
    grid=(GRID,),
    in_specs=[
        pl.BlockSpec(memory_space=pltpu.SMEM),
        pl.BlockSpec((BN, H), lambda i: (i, 0)),
        pl.BlockSpec((NUM_CORES, BN, H), lambda i: (0, i, 0)),
        _W_SPEC, _B_SPEC, _W_SPEC, _B_SPEC,
    ],
    out_specs=_H3_SPEC,
    out_shape=jax.ShapeDtypeStruct((T, NP, H), jnp.float32),
)

_mlp_mid = pl.pallas_call(
    _mlp_mid_body,
    grid=(GRID,),
    in_specs=[
        pl.BlockSpec(memory_space=pltpu.SMEM),
        _H3_SPEC,
        pl.BlockSpec((NUM_CORES, T, BN, H), lambda i: (0, 0, i, 0)),
        _W_SPEC, _B_SPEC, _W_SPEC, _B_SPEC,
    ],
    out_specs=_H3_SPEC,
    out_shape=jax.ShapeDtypeStruct((T, NP, H), jnp.float32),
)


def kernel(x, edge_index, W1, b1, W2, b2, eps):
    src = edge_index[0].reshape(NUM_WORKERS, EPW)
    dst = edge_index[1].reshape(NUM_WORKERS, CHUNKS, CHUNK)
    scale = 1.0 + eps
    zeros = jnp.zeros((CHUNK, H), jnp.float32)
    xp = jnp.pad(x, ((0, NP - N), (0, 0)))

    aggp0 = _sc_agg_1(xp[None], src, dst, zeros)
    h = _mlp_first(scale[:, 0], xp, aggp0[:, 0],
                   W1[:, 0], b1[:, 0], W2[:, 0], b2[:, 0])
    for l in range(1, L):
        aggp = _sc_agg_3(h, src, dst, zeros)
        h = _mlp_mid(scale[:, l], h, aggp,
                     W1[:, l], b1[:, l], W2[:, l], b2[:, l])
    return jnp.transpose(h[:, :N], (1, 0, 2))
